# Initial kernel scaffold; baseline (speedup 1.0000x reference)
import jax
import jax.numpy as jnp
from jax.experimental import pallas as pl


def kernel(input, get_hidden, train_hidden, train_hidden_day, W_ih0, W_hh0, b_ih0, b_hh0, W_ih1, W_hh1, b_ih1, b_hh1, lin0_W, lin0_b, lin1_W, lin1_b, lin2_W, lin2_b, p1_W, p2_W, fc_W, fc_b):
    raise NotImplementedError("write your pallas kernel here")


# speedup vs baseline: 1.8435x; 1.8435x over previous
import jax
import jax.numpy as jnp
from jax.experimental import pallas as pl
from jax.experimental.pallas import tpu as pltpu

H = 256
D_FEAT = 6
K_DAY = 10
N_NEIGHBOR = 10
T_SEQ = 20
NEG = -1e30


def _dot(a, b):
    return jnp.dot(a, b, preferred_element_type=jnp.float32)


def _gru_body(x_ref, bih0, bhh0, bih1, bhh1,
              wx0r, wx0z, wx0n, wh0r, wh0z, wh0n,
              wx1r, wx1z, wx1n, wh1r, wh1z, wh1n,
              out_ref):
    n = x_ref.shape[1]

    def layer(xt, h, wxr, wxz, wxn, whr, whz, whn, bih, bhh):
        gxr = _dot(xt, wxr[...]) + bih[0:1, 0:H]
        gxz = _dot(xt, wxz[...]) + bih[0:1, H:2 * H]
        gxn = _dot(xt, wxn[...]) + bih[0:1, 2 * H:]
        ghr = _dot(h, whr[...]) + bhh[0:1, 0:H]
        ghz = _dot(h, whz[...]) + bhh[0:1, H:2 * H]
        ghn = _dot(h, whn[...]) + bhh[0:1, 2 * H:]
        r = jax.nn.sigmoid(gxr + ghr)
        z = jax.nn.sigmoid(gxz + ghz)
        nn = jnp.tanh(gxn + r * ghn)
        return (1.0 - z) * nn + z * h

    def step(t, carry):
        h0, h1 = carry
        xt = x_ref[t]
        h0n = layer(xt, h0, wx0r, wx0z, wx0n, wh0r, wh0z, wh0n, bih0, bhh0)
        h1n = layer(h0n, h1, wx1r, wx1z, wx1n, wh1r, wh1z, wh1n, bih1, bhh1)
        return h0n, h1n

    h0 = jnp.zeros((n, H), jnp.float32)
    h1 = jnp.zeros((n, H), jnp.float32)
    h0, h1 = jax.lax.fori_loop(0, T_SEQ, step, (h0, h1))
    out_ref[...] = h1


def _mlp_body(h_ref, l0w, l0b, l1w, l1b, l2w, l2b, p1w, thd, thdT,
              mini_ref, p1_ref, didx_ref):
    def leaky(v):
        return jnp.where(v >= 0, v, 0.01 * v)

    m = leaky(_dot(h_ref[...], l0w[...]) + l0b[0:1, :])
    m = leaky(_dot(m, l1w[...]) + l1b[0:1, :])
    m = leaky(_dot(m, l2w[...]) + l2b[0:1, :])
    mini_ref[...] = m
    p1_ref[...] = _dot(m, p1w[...])

    nb = h_ref.shape[0] // 128
    md = jnp.concatenate(
        [jnp.mean(m[128 * d:128 * (d + 1), :], axis=0, keepdims=True)
         for d in range(nb)]
        + [jnp.zeros((8 - nb, m.shape[1]), jnp.float32)], axis=0)
    t = thd[...]
    tt = thdT[...]
    mdn = jnp.sqrt(jnp.sum(md * md, axis=1, keepdims=True))
    tn = jnp.sqrt(jnp.sum(tt * tt, axis=0, keepdims=True))
    xy = jax.lax.dot_general(md, t, (((1,), (1,)), ((), ())),
                             preferred_element_type=jnp.float32)
    ds = xy / (mdn * tn)
    ds = jnp.where(jnp.isnan(ds), 0.0, ds)

    lane = jax.lax.broadcasted_iota(jnp.int32, ds.shape, 1)
    slot = jax.lax.broadcasted_iota(jnp.int32, (ds.shape[0], 16), 1)
    idxs = jnp.zeros((ds.shape[0], 16), jnp.int32)
    for k in range(K_DAY):
        mx = jnp.max(ds, axis=1, keepdims=True)
        pos = jnp.min(jnp.where(ds == mx, lane, 128), axis=1, keepdims=True)
        idxs = jnp.where(slot == k, pos, idxs)
        ds = jnp.where(lane == pos, NEG, ds)
    didx_ref[...] = idxs


def _sim_body(didx, th_ref, p2w, p2wr, p1_ref, p2s_ref, cv_ref, ci_ref):
    i = pl.program_id(0)
    blk = th_ref[0]
    p2 = _dot(blk, p2w[...])
    p2s_ref[0] = p2
    p2t = jax.lax.dot_general(p2wr[...], blk, (((1,), (1,)), ((), ())),
                              preferred_element_type=jnp.float32)
    p1 = p1_ref[...]
    p1n = jnp.sqrt(jnp.sum(p1 * p1, axis=1, keepdims=True))
    rn = jnp.sqrt(jnp.sum(p2t * p2t, axis=0, keepdims=True))
    xy = jax.lax.dot_general(p1, p2, (((1,), (1,)), ((), ())),
                             preferred_element_type=jnp.float32)
    cs = xy / (p1n * rn)
    cs = jnp.where(jnp.isnan(cs), 0.0, cs)

    lane = jax.lax.broadcasted_iota(jnp.int32, cs.shape, 1)
    slot = jax.lax.broadcasted_iota(jnp.int32, (cs.shape[0], 16), 1)
    cv = jnp.full((cs.shape[0], 16), NEG, jnp.float32)
    ci = jnp.zeros((cs.shape[0], 16), jnp.int32)
    base = i * 256
    for k in range(N_NEIGHBOR):
        mx = jnp.max(cs, axis=1, keepdims=True)
        pos = jnp.min(jnp.where(cs == mx, lane, 256), axis=1, keepdims=True)
        cv = jnp.where(slot == k, mx, cv)
        ci = jnp.where(slot == k, pos + base, ci)
        cs = jnp.where(lane == pos, NEG, cs)
    cv_ref[0] = cv
    ci_ref[0] = ci


def _agg_body(cv_ref, ci_ref, p2s_ref, mini_ref, fcm, fca, fcb,
              o_ref, wv_s, wi_s, agg_s):
    i = pl.program_id(0)
    nblk = pl.num_programs(0)

    @pl.when(i == 0)
    def _():
        cv = cv_ref[...]
        ci = ci_ref[...]
        lane = jax.lax.broadcasted_iota(jnp.int32, cv.shape, 1)
        slot = jax.lax.broadcasted_iota(jnp.int32, (cv.shape[0], 16), 1)
        wv = jnp.zeros((cv.shape[0], 16), jnp.float32)
        wi = jnp.zeros((cv.shape[0], 16), jnp.int32)
        for k in range(N_NEIGHBOR):
            mx = jnp.max(cv, axis=1, keepdims=True)
            pos = jnp.min(jnp.where(cv == mx, lane, cv.shape[1]), axis=1, keepdims=True)
            gidx = jnp.sum(jnp.where(lane == pos, ci, 0), axis=1, keepdims=True)
            wv = jnp.where(slot == k, mx, wv)
            wi = jnp.where(slot == k, gidx, wi)
            cv = jnp.where(lane == pos, NEG, cv)
        wv_s[...] = wv
        wi_s[...] = wi
        agg_s[...] = jnp.zeros_like(agg_s)

    p2 = p2s_ref[0]
    lane256 = jax.lax.broadcasted_iota(jnp.int32, (wv_s.shape[0], 256), 1)
    w_blk = jnp.zeros((wv_s.shape[0], 256), jnp.float32)
    for k in range(N_NEIGHBOR):
        wcol = wi_s[:, k:k + 1] - i * 256
        val = wv_s[:, k:k + 1] / float(N_NEIGHBOR)
        w_blk = w_blk + jnp.where(lane256 == wcol, val, 0.0)
    agg = agg_s[...] + _dot(w_blk, p2)
    agg_s[...] = agg

    @pl.when(i == nblk - 1)
    def _():
        o = (_dot(mini_ref[...], fcm[...]) + _dot(agg, fca[...])
             + fcb[0:1, :])
        o_ref[...] = o


def kernel(input, get_hidden, train_hidden, train_hidden_day,
           W_ih0, W_hh0, b_ih0, b_hh0, W_ih1, W_hh1, b_ih1, b_hh1,
           lin0_W, lin0_b, lin1_W, lin1_b, lin2_W, lin2_b,
           p1_W, p2_W, fc_W, fc_b):
    B, S, _ = input.shape
    N = B * S
    x = input.reshape(N, D_FEAT, -1)
    x = jnp.transpose(x, (2, 0, 1))
    x = jnp.pad(x, ((0, 0), (0, 0), (0, 8 - D_FEAT)))

    def gate_w(W, K, pad_to=None):
        r, z, n = W[:H].T, W[H:2 * H].T, W[2 * H:].T
        if pad_to is not None:
            r = jnp.pad(r, ((0, pad_to - K), (0, 0)))
            z = jnp.pad(z, ((0, pad_to - K), (0, 0)))
            n = jnp.pad(n, ((0, pad_to - K), (0, 0)))
        return r, z, n

    ws = (*gate_w(W_ih0, D_FEAT, pad_to=8), *gate_w(W_hh0, H),
          *gate_w(W_ih1, H), *gate_w(W_hh1, H))
    bs = (b_ih0.reshape(1, -1), b_hh0.reshape(1, -1),
          b_ih1.reshape(1, -1), b_hh1.reshape(1, -1))

    h1 = pl.pallas_call(
        _gru_body,
        out_shape=jax.ShapeDtypeStruct((N, H), jnp.float32),
    )(x, *bs, *ws)

    mini, p1, didx = pl.pallas_call(
        _mlp_body,
        out_shape=(
            jax.ShapeDtypeStruct((N, H), jnp.float32),
            jax.ShapeDtypeStruct((N, H), jnp.float32),
            jax.ShapeDtypeStruct((8, 16), jnp.int32),
        ),
    )(h1, lin0_W.T, lin0_b.reshape(1, -1), lin1_W.T, lin1_b.reshape(1, -1),
      lin2_W.T, lin2_b.reshape(1, -1), p1_W.T, train_hidden_day,
      train_hidden_day.T)

    day_flat = didx[:B, :K_DAY].reshape(-1)
    nblk = B * K_DAY

    p2s, cand_v, cand_i = pl.pallas_call(
        _sim_body,
        grid_spec=pltpu.PrefetchScalarGridSpec(
            num_scalar_prefetch=1,
            grid=(nblk,),
            in_specs=[
                pl.BlockSpec((1, 256, H), lambda i, didx_ref: (didx_ref[i], 0, 0)),
                pl.BlockSpec((H, H), lambda i, didx_ref: (0, 0)),
                pl.BlockSpec((H, H), lambda i, didx_ref: (0, 0)),
                pl.BlockSpec((N, H), lambda i, didx_ref: (0, 0)),
            ],
            out_specs=[
                pl.BlockSpec((1, 256, H), lambda i, didx_ref: (i, 0, 0)),
                pl.BlockSpec((1, N, 16), lambda i, didx_ref: (i, 0, 0)),
                pl.BlockSpec((1, N, 16), lambda i, didx_ref: (i, 0, 0)),
            ],
        ),
        out_shape=(
            jax.ShapeDtypeStruct((nblk, 256, H), jnp.float32),
            jax.ShapeDtypeStruct((nblk, N, 16), jnp.float32),
            jax.ShapeDtypeStruct((nblk, N, 16), jnp.int32),
        ),
    )(day_flat, train_hidden, p2_W.T, p2_W, p1)

    cv_flat = jnp.transpose(cand_v, (1, 0, 2)).reshape(N, nblk * 16)
    ci_flat = jnp.transpose(cand_i, (1, 0, 2)).reshape(N, nblk * 16)

    o = pl.pallas_call(
        _agg_body,
        grid=(nblk,),
        in_specs=[
            pl.BlockSpec((N, nblk * 16), lambda i: (0, 0)),
            pl.BlockSpec((N, nblk * 16), lambda i: (0, 0)),
            pl.BlockSpec((1, 256, H), lambda i: (i, 0, 0)),
            pl.BlockSpec((N, H), lambda i: (0, 0)),
            pl.BlockSpec((H, 1), lambda i: (0, 0)),
            pl.BlockSpec((H, 1), lambda i: (0, 0)),
            pl.BlockSpec((1, 1), lambda i: (0, 0)),
        ],
        out_specs=pl.BlockSpec((N, 1), lambda i: (0, 0)),
        out_shape=jax.ShapeDtypeStruct((N, 1), jnp.float32),
        scratch_shapes=[
            pltpu.VMEM((N, 16), jnp.float32),
            pltpu.VMEM((N, 16), jnp.int32),
            pltpu.VMEM((N, H), jnp.float32),
        ],
    )(cv_flat, ci_flat, p2s, mini, fc_W[:, :H].T, fc_W[:, H:].T,
      fc_b.reshape(1, 1))

    return o.reshape(B, S)
